# baseline (speedup 1.0000x reference)
import functools

import jax
import jax.numpy as jnp
from jax import lax
from jax.experimental import pallas as pl
from jax.experimental.pallas import tpu as pltpu
from jax.experimental.pallas import tpu_sc as plsc

D = 128
C = 64
B = 4096
L = 50
NC, NS = 2, 16
NW = NC * NS
SPW = B // NW
NV = D // 16

_sc_mesh = plsc.VectorSubcoreMesh(core_axis_name="c", subcore_axis_name="s")


@functools.partial(
    pl.kernel,
    out_type=jax.ShapeDtypeStruct((B, D), jnp.float32),
    mesh=_sc_mesh,
    scratch_types=[
        pltpu.VMEM((SPW // 2, 2 * L), jnp.int32),
        pltpu.VMEM((4, 2 * L, D), jnp.float32),
        pltpu.VMEM((SPW, D), jnp.float32),
        pltpu.SemaphoreType.DMA,
        pltpu.SemaphoreType.DMA,
        pltpu.SemaphoreType.DMA,
        pltpu.SemaphoreType.DMA,
        pltpu.SemaphoreType.DMA,
    ],
)
def _sc_pool(idx_hbm, table_hbm, out_hbm, idx_v, rows_v, out_v,
             sem0, sem1, sem2, sem3, out_sem):
    wid = lax.axis_index("s") * NC + lax.axis_index("c")
    nch = SPW // 2
    sems = (sem0, sem1, sem2, sem3)
    nbuf = 4

    pltpu.sync_copy(idx_hbm.at[pl.ds(wid * nch, nch)], idx_v)

    for bf in range(nbuf):
        pltpu.async_copy(table_hbm.at[idx_v.at[bf]], rows_v.at[bf], sems[bf])

    def body(g, carry):
        for bf in range(nbuf):
            c = nbuf * g + bf
            pltpu.make_async_copy(
                table_hbm.at[idx_v.at[0]], rows_v.at[bf], sems[bf]
            ).wait()
            for h in range(2):
                r0 = h * L

                def rbody(r, accs):
                    return tuple(
                        accs[d] + rows_v[bf, r, pl.ds(16 * d, 16)]
                        for d in range(NV)
                    )

                accs0 = tuple(
                    rows_v[bf, r0, pl.ds(16 * d, 16)] for d in range(NV)
                )
                accs = lax.fori_loop(r0 + 1, r0 + L, rbody, accs0)
                for d in range(NV):
                    out_v[2 * c + h, pl.ds(16 * d, 16)] = accs[d]

            nxt = c + nbuf

            @pl.when(nxt < nch)
            def _():
                pltpu.async_copy(table_hbm.at[idx_v.at[nxt]], rows_v.at[bf], sems[bf])

        return carry

    half = (nch // nbuf) // 2
    lax.fori_loop(0, half, body, 0)
    pltpu.async_copy(
        out_v.at[pl.ds(0, SPW // 2)],
        out_hbm.at[pl.ds(wid * SPW, SPW // 2)], out_sem,
    )
    lax.fori_loop(half, nch // nbuf, body, 0)
    pltpu.sync_copy(
        out_v.at[pl.ds(SPW // 2, SPW // 2)],
        out_hbm.at[pl.ds(wid * SPW + SPW // 2, SPW // 2)],
    )
    pltpu.make_async_copy(
        out_v.at[pl.ds(0, SPW // 2)],
        out_hbm.at[pl.ds(wid * SPW, SPW // 2)], out_sem,
    ).wait()


def _head_body(x_ref, w_ref, b_ref, o_ref):
    logits = lax.dot_general(
        x_ref[...], w_ref[...], (((1,), (1,)), ((), ())),
        preferred_element_type=jnp.float32,
    ) + b_ref[...]
    m = jnp.max(logits, axis=-1, keepdims=True)
    shifted = logits - m
    denom = jnp.sum(jnp.exp(shifted), axis=-1, keepdims=True)
    o_ref[...] = shifted - jnp.log(denom)


def _tc_head(pooled, W, b2):
    g = 1
    bb = B // g
    return pl.pallas_call(
        _head_body,
        grid=(g,),
        in_specs=[
            pl.BlockSpec((bb, D), lambda i: (i, 0)),
            pl.BlockSpec((C, D), lambda i: (0, 0)),
            pl.BlockSpec((1, C), lambda i: (0, 0)),
        ],
        out_specs=pl.BlockSpec((bb, C), lambda i: (i, 0)),
        out_shape=jax.ShapeDtypeStruct((B, C), jnp.float32),
    )(pooled, W, b2)


def kernel(inputs, table, W, b):
    idx = inputs.astype(jnp.int32).reshape(B // 2, 2 * L)
    pooled = _sc_pool(idx, table)
    return _tc_head(pooled, W, b.reshape(1, C))


# speedup vs baseline: 1.0091x; 1.0091x over previous
import functools

import jax
import jax.numpy as jnp
from jax import lax
from jax.experimental import pallas as pl
from jax.experimental.pallas import tpu as pltpu
from jax.experimental.pallas import tpu_sc as plsc

D = 128
C = 64
B = 4096
L = 50
NC, NS = 2, 16
NW = NC * NS
SPW = B // NW
NV = D // 16

_sc_mesh = plsc.VectorSubcoreMesh(core_axis_name="c", subcore_axis_name="s")


@functools.partial(
    pl.kernel,
    out_type=jax.ShapeDtypeStruct((B, D), jnp.float32),
    mesh=_sc_mesh,
    scratch_types=[
        pltpu.VMEM((SPW // 2, 2 * L), jnp.int32),
        pltpu.VMEM((4, 2 * L, D), jnp.float32),
        pltpu.VMEM((SPW, D), jnp.float32),
        pltpu.SemaphoreType.DMA,
        pltpu.SemaphoreType.DMA,
        pltpu.SemaphoreType.DMA,
        pltpu.SemaphoreType.DMA,
    ],
)
def _sc_pool(idx_hbm, table_hbm, out_hbm, idx_v, rows_v, out_v,
             sem0, sem1, sem2, sem3):
    wid = lax.axis_index("s") * NC + lax.axis_index("c")
    nch = SPW // 2
    sems = (sem0, sem1, sem2, sem3)
    nbuf = 4

    pltpu.sync_copy(idx_hbm.at[pl.ds(wid * nch, nch)], idx_v)

    for bf in range(nbuf):
        pltpu.async_copy(table_hbm.at[idx_v.at[bf]], rows_v.at[bf], sems[bf])

    def body(g, carry):
        for bf in range(nbuf):
            c = nbuf * g + bf
            pltpu.make_async_copy(
                table_hbm.at[idx_v.at[0]], rows_v.at[bf], sems[bf]
            ).wait()
            for h in range(2):
                r0 = h * L

                def rbody(r, accs):
                    return tuple(
                        accs[d] + rows_v[bf, r, pl.ds(16 * d, 16)]
                        for d in range(NV)
                    )

                accs0 = tuple(
                    rows_v[bf, r0, pl.ds(16 * d, 16)] for d in range(NV)
                )
                accs = lax.fori_loop(r0 + 1, r0 + L, rbody, accs0)
                for d in range(NV):
                    out_v[2 * c + h, pl.ds(16 * d, 16)] = accs[d]

            nxt = c + nbuf

            @pl.when(nxt < nch)
            def _():
                pltpu.async_copy(table_hbm.at[idx_v.at[nxt]], rows_v.at[bf], sems[bf])

        return carry

    lax.fori_loop(0, nch // nbuf, body, 0)
    pltpu.sync_copy(out_v, out_hbm.at[pl.ds(wid * SPW, SPW)])


def _head_body(x_ref, w_ref, b_ref, o_ref):
    logits = lax.dot_general(
        x_ref[...], w_ref[...], (((1,), (1,)), ((), ())),
        preferred_element_type=jnp.float32,
    ) + b_ref[...]
    m = jnp.max(logits, axis=-1, keepdims=True)
    shifted = logits - m
    denom = jnp.sum(jnp.exp(shifted), axis=-1, keepdims=True)
    o_ref[...] = shifted - jnp.log(denom)


def _tc_head(pooled, W, b2):
    g = 1
    bb = B // g
    return pl.pallas_call(
        _head_body,
        grid=(g,),
        in_specs=[
            pl.BlockSpec((bb, D), lambda i: (i, 0)),
            pl.BlockSpec((C, D), lambda i: (0, 0)),
            pl.BlockSpec((1, C), lambda i: (0, 0)),
        ],
        out_specs=pl.BlockSpec((bb, C), lambda i: (i, 0)),
        out_shape=jax.ShapeDtypeStruct((B, C), jnp.float32),
    )(pooled, W, b2)


def kernel(inputs, table, W, b):
    idx = inputs.astype(jnp.int32).reshape(B // 2, 2 * L)
    pooled = _sc_pool(idx, table)
    return _tc_head(pooled, W, b.reshape(1, C))
